# baseline (speedup 1.0000x reference)
import functools

import jax
import jax.numpy as jnp
from jax import lax
from jax.experimental import pallas as pl
from jax.experimental.pallas import tpu as pltpu
from jax.experimental.pallas import tpu_sc as plsc

NC = 2
NS = 16
L = 16
NBUF = 2


@functools.lru_cache(maxsize=None)
def _build(M, D, B):
    MC = 7168
    NCH = -(-M // MC)
    NCH += NCH % NC
    KPC = NCH // NC
    RPT = MC // NS
    BPT = B // NS
    G = 64
    GSH = 6
    NG = (BPT + G) // G
    assert BPT * NS == B and BPT % L == 0 and BPT % G == 0
    assert (NCH - 1) * MC >= M - MC
    assert M - MC >= 0 and (M - MC) % 8 == 0

    mesh = plsc.VectorSubcoreMesh(
        core_axis_name="c", subcore_axis_name="s",
        num_cores=NC, num_subcores=NS)

    @functools.partial(
        pl.kernel,
        out_type=jax.ShapeDtypeStruct((M, D), jnp.float32),
        mesh=mesh,
        compiler_params=pltpu.CompilerParams(needs_layout_passes=False),
        scratch_types=(
            [pltpu.VMEM_SHARED((MC + NS, D), jnp.float32)
             for _ in range(NBUF)] +
            [
                pltpu.VMEM((BPT,), jnp.int32),
                pltpu.VMEM((NG, G), jnp.int32),
                pltpu.VMEM((NG, G), jnp.int32),
                pltpu.VMEM((G, D), jnp.float32),
                pltpu.VMEM((L,), jnp.float32),
            ] +
            [pltpu.SemaphoreType.DMA for _ in range(NBUF)] +
            [pltpu.SemaphoreType.DMA for _ in range(NBUF)] +
            [pltpu.SemaphoreType.DMA]
        ),
    )
    def _ker(x_hbm, idx_hbm, src_hbm, alpha_hbm, out_hbm, *scr):
        accs = scr[:NBUF]
        idx_v, selb_v, selr_v, gsrc_v, alpha_v = scr[NBUF:NBUF + 5]
        semls = scr[NBUF + 5:2 * NBUF + 5]
        semws = scr[2 * NBUF + 5:3 * NBUF + 5]
        semg = scr[3 * NBUF + 5]

        c = lax.axis_index("c")
        s = lax.axis_index("s")
        trash = MC + s
        sbase = pl.multiple_of(s * RPT, 8)

        pltpu.sync_copy(idx_hbm.at[pl.ds(pl.multiple_of(s * BPT, 8), BPT)],
                        idx_v)
        pltpu.sync_copy(alpha_hbm, alpha_v)
        av = alpha_v[...]
        noscale = av[0] == jnp.float32(1.0)

        def chunk_lo(k):
            return jnp.minimum((KPC * c + k) * MC, M - MC)

        def tile_base(lo):
            return pl.multiple_of(lo + s * RPT, 8)

        def issue_load(k, p):
            return pltpu.async_copy(
                x_hbm.at[pl.ds(tile_base(chunk_lo(k)), RPT)],
                accs[p].at[pl.ds(sbase, RPT)], semls[p])

        wb_desc = [None] * NBUF
        load_desc = [None] * NBUF
        load_desc[0] = issue_load(0, 0)

        for k in range(KPC):
            p = k % NBUF
            acc = accs[p]
            lo = chunk_lo(k)
            hi = lo + MC

            zero_v = jnp.zeros((L,), jnp.int32)

            def sel_body(j, cnt):
                v = idx_v[pl.ds(j * L, L)]
                m = (v >= lo) & (v < hi)
                mi = jnp.where(m, jnp.int32(1), jnp.int32(0))
                bsrc = s * BPT + j * L + lax.iota(jnp.int32, L)
                ps = jnp.cumsum(mi)
                pos = cnt + ps - 1
                pg = jnp.right_shift(pos, GSH)
                po = jnp.bitwise_and(pos, G - 1)
                plsc.store_scatter(selb_v, [pg, po], bsrc, mask=m)
                plsc.store_scatter(selr_v, [pg, po], v - lo, mask=m)
                return cnt + jnp.sum(mi)

            cnt = lax.fori_loop(0, BPT // L, sel_body, jnp.int32(0))

            trash_v = jnp.broadcast_to(trash, (L,)).astype(jnp.int32)
            for t in range(G // L):
                pp = cnt + t * L + lax.iota(jnp.int32, L)
                pg = jnp.right_shift(pp, GSH)
                po = jnp.bitwise_and(pp, G - 1)
                pad_b = s * BPT + t * L + lax.iota(jnp.int32, L)
                plsc.store_scatter(selb_v, [pg, po], pad_b)
                plsc.store_scatter(selr_v, [pg, po], trash_v)
            ng = (cnt + G - 1) // G

            gd = pltpu.async_copy(src_hbm.at[selb_v.at[0]], gsrc_v, semg)

            load_desc[p].wait()

            if k + 1 < KPC:
                q = (k + 1) % NBUF
                if wb_desc[q] is not None:
                    wb_desc[q].wait()
                load_desc[q] = issue_load(k + 1, q)

            plsc.subcore_barrier()
            gd.wait()

            def proc_body(g, _):
                @pl.when(g > 0)
                def _():
                    pltpu.sync_copy(src_hbm.at[selb_v.at[g]], gsrc_v)

                @pl.when(jnp.logical_not(noscale))
                def _():
                    def scale_row(r, _):
                        for cb in range(D // L):
                            sl = pl.ds(cb * L, L)
                            gsrc_v[r, sl] = gsrc_v[r, sl] * av
                        return 0
                    lax.fori_loop(0, G, scale_row, 0)

                pltpu.sync_copy(gsrc_v, acc.at[selr_v.at[g]], add=True)
                return 0

            lax.fori_loop(0, ng, proc_body, 0)
            plsc.subcore_barrier()

            wb_desc[p] = pltpu.async_copy(
                acc.at[pl.ds(sbase, RPT)],
                out_hbm.at[pl.ds(tile_base(lo), RPT)], semws[p])

        for d in wb_desc:
            if d is not None:
                d.wait()

    return _ker


def kernel(x, dim, index, source, alpha, out):
    M, D = x.shape
    B = index.shape[0]
    alpha_arr = jnp.full((L,), alpha, jnp.float32)
    return _build(M, D, B)(x, index.astype(jnp.int32), source, alpha_arr)


# speedup vs baseline: 1.1353x; 1.1353x over previous
import functools

import jax
import jax.numpy as jnp
from jax import lax
from jax.experimental import pallas as pl
from jax.experimental.pallas import tpu as pltpu
from jax.experimental.pallas import tpu_sc as plsc

NC = 2
NS = 16
L = 16
NBUF = 3


@functools.lru_cache(maxsize=None)
def _build(M, D, B):
    MC = 4736
    NCH = -(-M // MC)
    NCH += NCH % NC
    KPC = NCH // NC
    RPT = MC // NS
    BPT = B // NS
    G = 64
    GSH = 6
    NG = (BPT + G) // G
    assert BPT * NS == B and BPT % L == 0 and BPT % G == 0
    assert (NCH - 1) * MC >= M - MC
    assert M - MC >= 0 and (M - MC) % 8 == 0

    mesh = plsc.VectorSubcoreMesh(
        core_axis_name="c", subcore_axis_name="s",
        num_cores=NC, num_subcores=NS)

    @functools.partial(
        pl.kernel,
        out_type=jax.ShapeDtypeStruct((M, D), jnp.float32),
        mesh=mesh,
        compiler_params=pltpu.CompilerParams(needs_layout_passes=False),
        scratch_types=(
            [pltpu.VMEM_SHARED((MC + NS, D), jnp.float32)
             for _ in range(NBUF)] +
            [
                pltpu.VMEM((BPT,), jnp.int32),
                pltpu.VMEM((NG, G), jnp.int32),
                pltpu.VMEM((NG, G), jnp.int32),
                pltpu.VMEM((G, D), jnp.float32),
                pltpu.VMEM((L,), jnp.float32),
            ] +
            [pltpu.SemaphoreType.DMA for _ in range(NBUF)] +
            [pltpu.SemaphoreType.DMA for _ in range(NBUF)] +
            [pltpu.SemaphoreType.DMA]
        ),
    )
    def _ker(x_hbm, idx_hbm, src_hbm, alpha_hbm, out_hbm, *scr):
        accs = scr[:NBUF]
        idx_v, selb_v, selr_v, gsrc_v, alpha_v = scr[NBUF:NBUF + 5]
        semls = scr[NBUF + 5:2 * NBUF + 5]
        semws = scr[2 * NBUF + 5:3 * NBUF + 5]
        semg = scr[3 * NBUF + 5]

        c = lax.axis_index("c")
        s = lax.axis_index("s")
        trash = MC + s
        sbase = pl.multiple_of(s * RPT, 8)

        pltpu.sync_copy(idx_hbm.at[pl.ds(pl.multiple_of(s * BPT, 8), BPT)],
                        idx_v)
        pltpu.sync_copy(alpha_hbm, alpha_v)
        av = alpha_v[...]
        noscale = av[0] == jnp.float32(1.0)

        def chunk_lo(k):
            return jnp.minimum((KPC * c + k) * MC, M - MC)

        def tile_base(lo):
            return pl.multiple_of(lo + s * RPT, 8)

        def issue_load(k, p):
            return pltpu.async_copy(
                x_hbm.at[pl.ds(tile_base(chunk_lo(k)), RPT)],
                accs[p].at[pl.ds(sbase, RPT)], semls[p])

        def issue_wb(k, ):
            p = k % NBUF
            lo = chunk_lo(k)
            return pltpu.async_copy(
                accs[p].at[pl.ds(sbase, RPT)],
                out_hbm.at[pl.ds(tile_base(lo), RPT)], semws[p])

        wb_desc = [None] * NBUF
        load_desc = [None] * NBUF
        load_desc[0] = issue_load(0, 0)

        for k in range(KPC):
            p = k % NBUF
            acc = accs[p]
            lo = chunk_lo(k)
            hi = lo + MC

            zero_v = jnp.zeros((L,), jnp.int32)

            def sel_body(j, cnt):
                v = idx_v[pl.ds(j * L, L)]
                m = (v >= lo) & (v < hi)
                mi = jnp.where(m, jnp.int32(1), jnp.int32(0))
                bsrc = s * BPT + j * L + lax.iota(jnp.int32, L)
                ps = jnp.cumsum(mi)
                pos = cnt + ps - 1
                pg = jnp.right_shift(pos, GSH)
                po = jnp.bitwise_and(pos, G - 1)
                plsc.store_scatter(selb_v, [pg, po], bsrc, mask=m)
                plsc.store_scatter(selr_v, [pg, po], v - lo, mask=m)
                return cnt + jnp.sum(mi)

            cnt = lax.fori_loop(0, BPT // L, sel_body, jnp.int32(0))

            trash_v = jnp.broadcast_to(trash, (L,)).astype(jnp.int32)
            for t in range(G // L):
                pp = cnt + t * L + lax.iota(jnp.int32, L)
                pg = jnp.right_shift(pp, GSH)
                po = jnp.bitwise_and(pp, G - 1)
                pad_b = s * BPT + t * L + lax.iota(jnp.int32, L)
                plsc.store_scatter(selb_v, [pg, po], pad_b)
                plsc.store_scatter(selr_v, [pg, po], trash_v)
            ng = (cnt + G - 1) // G

            gd = pltpu.async_copy(src_hbm.at[selb_v.at[0]], gsrc_v, semg)

            load_desc[p].wait()

            if k + 1 < KPC:
                q = (k + 1) % NBUF
                if wb_desc[q] is not None:
                    wb_desc[q].wait()
                load_desc[q] = issue_load(k + 1, q)

            plsc.subcore_barrier()

            if k > 0:
                wb_desc[(k - 1) % NBUF] = issue_wb(k - 1)
            gd.wait()

            def proc_body(g, _):
                @pl.when(g > 0)
                def _():
                    pltpu.sync_copy(src_hbm.at[selb_v.at[g]], gsrc_v)

                @pl.when(jnp.logical_not(noscale))
                def _():
                    def scale_row(r, _):
                        for cb in range(D // L):
                            sl = pl.ds(cb * L, L)
                            gsrc_v[r, sl] = gsrc_v[r, sl] * av
                        return 0
                    lax.fori_loop(0, G, scale_row, 0)

                pltpu.sync_copy(gsrc_v, acc.at[selr_v.at[g]], add=True)
                return 0

            lax.fori_loop(0, ng, proc_body, 0)

        plsc.subcore_barrier()
        wb_desc[(KPC - 1) % NBUF] = issue_wb(KPC - 1)
        for d in wb_desc:
            if d is not None:
                d.wait()

    return _ker


def kernel(x, dim, index, source, alpha, out):
    M, D = x.shape
    B = index.shape[0]
    alpha_arr = jnp.full((L,), alpha, jnp.float32)
    return _build(M, D, B)(x, index.astype(jnp.int32), source, alpha_arr)
